# Initial kernel scaffold; baseline (speedup 1.0000x reference)
import jax
import jax.numpy as jnp
from jax.experimental import pallas as pl


def kernel(x, edge_index, W0, W1, W2, b):
    raise NotImplementedError("write your pallas kernel here")


# speedup vs baseline: 28.1862x; 28.1862x over previous
import functools

import jax
import jax.numpy as jnp
from jax import lax
from jax.experimental import pallas as pl
from jax.experimental.pallas import tpu as pltpu
from jax.experimental.pallas import tpu_sc as plsc

NC = 2
NS = 16
NW = NC * NS
LANES = 16
CH = 128


def _sc_mesh():
    return plsc.VectorSubcoreMesh(
        core_axis_name="c", subcore_axis_name="s",
        num_cores=NC, num_subcores=NS)


_SC_PARAMS = pltpu.CompilerParams(use_tc_tiling_on_sc=False)


def _make_sc_deg(npad, k_rows):
    rps = npad // NS

    @functools.partial(
        pl.kernel,
        out_type=jax.ShapeDtypeStruct((NC, npad), jnp.float32),
        mesh=_sc_mesh(),
        compiler_params=_SC_PARAMS,
        scratch_types=[
            pltpu.VMEM((k_rows, CH), jnp.int32),
            pltpu.VMEM((CH,), jnp.float32),
            pltpu.VMEM((rps,), jnp.float32),
            pltpu.VMEM_SHARED((npad,), jnp.float32),
            pltpu.SemaphoreType.DMA,
        ],
    )
    def sc_deg(rowp_hbm, out_hbm, idx_v, ones_v, zbuf_v, acc, sem):
        c = lax.axis_index("c")
        s = lax.axis_index("s")
        wid = c * NS + s
        pltpu.sync_copy(rowp_hbm.at[wid], idx_v)
        for j in range(CH // LANES):
            ones_v[pl.ds(j * LANES, LANES)] = jnp.ones((LANES,), jnp.float32)
        for j in range(rps // LANES):
            zbuf_v[pl.ds(j * LANES, LANES)] = jnp.zeros((LANES,), jnp.float32)
        pltpu.sync_copy(zbuf_v, acc.at[pl.ds(s * rps, rps)])
        plsc.subcore_barrier()

        def body(k, carry):
            pltpu.sync_copy(ones_v, acc.at[idx_v.at[k]], add=True)
            return carry

        lax.fori_loop(0, k_rows, body, 0)
        plsc.subcore_barrier()
        pltpu.sync_copy(acc.at[pl.ds(s * rps, rps)],
                        out_hbm.at[c, pl.ds(s * rps, rps)])

    return sc_deg


def _make_sc_pass(n_src, npad, k_rows):
    rps = npad // NS

    @functools.partial(
        pl.kernel,
        out_type=jax.ShapeDtypeStruct((NC, npad, LANES), jnp.float32),
        mesh=_sc_mesh(),
        compiler_params=_SC_PARAMS,
        scratch_types=[
            pltpu.VMEM((k_rows, CH), jnp.int32),
            pltpu.VMEM((k_rows, CH), jnp.int32),
            pltpu.VMEM((CH, LANES), jnp.float32),
            pltpu.VMEM((CH, LANES), jnp.float32),
            pltpu.VMEM_SHARED((npad, LANES), jnp.float32),
            pltpu.SemaphoreType.DMA,
        ],
    )
    def sc_pass(src_hbm, gidx_hbm, sidx_hbm, out_hbm,
                gidx_v, sidx_v, rows_v, zrow_v, acc, gsem):
        c = lax.axis_index("c")
        s = lax.axis_index("s")
        wid = c * NS + s
        pltpu.sync_copy(gidx_hbm.at[wid], gidx_v)
        pltpu.sync_copy(sidx_hbm.at[wid], sidx_v)
        for j in range(CH):
            zrow_v[j] = jnp.zeros((LANES,), jnp.float32)
        for t in range(rps // CH):
            pltpu.sync_copy(zrow_v, acc.at[pl.ds(s * rps + t * CH, CH)])
        plsc.subcore_barrier()

        def body(k, carry):
            pltpu.async_copy(src_hbm.at[gidx_v.at[k]], rows_v, gsem).wait()
            pltpu.sync_copy(rows_v, acc.at[sidx_v.at[k]], add=True)
            return carry

        lax.fori_loop(0, k_rows, body, 0)
        plsc.subcore_barrier()
        pltpu.sync_copy(acc.at[pl.ds(s * rps, rps)],
                        out_hbm.at[c, pl.ds(s * rps, rps)])

    return sc_pass


def _dinv_col(deg_ref):
    degb = deg_ref[...]
    deg = degb[:, 0:1] + degb[:, 1:2]
    return jnp.where(deg > 0, lax.rsqrt(jnp.maximum(deg, 1e-12)), 0.0)


def _tc_pre(x, wcat, b2, deg_part, bn):
    n, d = x.shape
    npad = deg_part.shape[1]

    def kern(x_ref, w_ref, b_ref, deg_ref, ys_ref, p_ref):
        yp = jnp.dot(x_ref[...], w_ref[...],
                     precision=lax.Precision.HIGHEST,
                     preferred_element_type=jnp.float32)
        dinv_c = _dinv_col(deg_ref)
        ys_ref[...] = dinv_c * yp[:, :LANES]
        p_ref[...] = yp[:, LANES:LANES + 8] + b_ref[...]

    return pl.pallas_call(
        kern,
        grid=(n // bn,),
        in_specs=[
            pl.BlockSpec((bn, d), lambda i: (i, 0)),
            pl.BlockSpec((d, 24), lambda i: (0, 0)),
            pl.BlockSpec((1, 8), lambda i: (0, 0)),
            pl.BlockSpec((bn, NC), lambda i: (i, 0)),
        ],
        out_specs=[
            pl.BlockSpec((bn, LANES), lambda i: (i, 0)),
            pl.BlockSpec((bn, 8), lambda i: (i, 0)),
        ],
        out_shape=[
            jax.ShapeDtypeStruct((n, LANES), jnp.float32),
            jax.ShapeDtypeStruct((n, 8), jnp.float32),
        ],
    )(x, wcat, b2, deg_part)


def _tc_mid(s_part, deg_part, n, bn):
    def kern(s_ref, deg_ref, zs_ref, zc_ref):
        sb = s_ref[...]
        stot = sb[0] + sb[1]
        dinv_c = _dinv_col(deg_ref)
        zs_ref[...] = -(dinv_c * dinv_c) * stot
        zc_ref[...] = -dinv_c * stot[:, :8]

    return pl.pallas_call(
        kern,
        grid=(n // bn,),
        in_specs=[
            pl.BlockSpec((NC, bn, LANES), lambda i: (0, i, 0)),
            pl.BlockSpec((bn, NC), lambda i: (i, 0)),
        ],
        out_specs=[
            pl.BlockSpec((bn, LANES), lambda i: (i, 0)),
            pl.BlockSpec((bn, 8), lambda i: (i, 0)),
        ],
        out_shape=[
            jax.ShapeDtypeStruct((n, LANES), jnp.float32),
            jax.ShapeDtypeStruct((n, 8), jnp.float32),
        ],
    )(s_part, deg_part)


def _tc_fin(p, zc, s2_part, deg_part, n, bn):
    def kern(p_ref, zc_ref, s_ref, deg_ref, o_ref):
        sb = s_ref[...]
        stot = sb[0] + sb[1]
        dinv_c = _dinv_col(deg_ref)
        zz = -dinv_c * stot[:, 8:]
        o_ref[...] = jnp.maximum(p_ref[...] + zc_ref[...] + 2.0 * zz, 0.0)

    return pl.pallas_call(
        kern,
        grid=(n // bn,),
        in_specs=[
            pl.BlockSpec((bn, 8), lambda i: (i, 0)),
            pl.BlockSpec((bn, 8), lambda i: (i, 0)),
            pl.BlockSpec((NC, bn, LANES), lambda i: (0, i, 0)),
            pl.BlockSpec((bn, NC), lambda i: (i, 0)),
        ],
        out_specs=pl.BlockSpec((bn, 8), lambda i: (i, 0)),
        out_shape=jax.ShapeDtypeStruct((n, 8), jnp.float32),
    )(p, zc, s2_part, deg_part)


def kernel(x, edge_index, W0, W1, W2, b):
    n, d = x.shape
    out_f = W0.shape[1]
    e = edge_index.shape[1]

    k_rows = -(-e // (NW * CH))
    e_pad = NW * CH * k_rows
    npad = -(-(n + LANES) // (NS * CH)) * (NS * CH)
    bn = 2000
    assert n % bn == 0 and npad % (NS * CH) == 0

    row = edge_index[0]
    col = edge_index[1]
    spread = n + (jnp.arange(e, dtype=jnp.int32) & (LANES - 1))
    slm = row == col
    rowp = jnp.where(slm, spread, row)
    colp = jnp.where(slm, spread, col)
    pad_n = e_pad - e
    pad_scatter = n + (jnp.arange(pad_n, dtype=jnp.int32) & (LANES - 1))
    pad_gather = (jnp.arange(pad_n, dtype=jnp.int32) * 997) % n

    shape3 = (NW, k_rows, CH)
    rowp3 = jnp.concatenate([rowp, pad_scatter]).reshape(shape3)
    colp3 = jnp.concatenate([colp, pad_scatter]).reshape(shape3)
    rowg3 = jnp.concatenate([row, pad_gather]).reshape(shape3)

    deg_part = _make_sc_deg(npad, k_rows)(rowp3)
    deg_t = jnp.transpose(deg_part)

    wcat = jnp.concatenate([W1, W2, W0 - W2], axis=1)
    b2 = b.reshape(1, out_f)
    ys, p = _tc_pre(x, wcat, b2, deg_t, bn)

    sc_pass = _make_sc_pass(n, npad, k_rows)
    s_part = sc_pass(ys, rowg3, colp3)
    zs, zc = _tc_mid(s_part, deg_t, n, bn)
    s2_part = sc_pass(zs, rowg3, colp3)
    return _tc_fin(p, zc, s2_part, deg_t, n, bn)
